# baseline (speedup 1.0000x reference)
import functools

import jax
import jax.numpy as jnp
from jax import lax
from jax.experimental import pallas as pl
from jax.experimental.pallas import tpu as pltpu
from jax.experimental.pallas import tpu_sc as plsc

DIM = 64
BATCH = 16384
LANES = 16
NC = 2
NS = 16
NW = NC * NS
B_PER_W = BATCH // NW
CHUNK = 128
NCHUNK = B_PER_W // CHUNK


def _body(ent_hbm, rel_hbm, hidx_hbm, tidx_hbm, ridx_hbm, out_hbm,
          hij, tij, rij, hbuf, tbuf, rbuf, sem):
  wid = lax.axis_index("s") * NC + lax.axis_index("c")
  pltpu.sync_copy(hidx_hbm.at[wid], hij)
  pltpu.sync_copy(tidx_hbm.at[wid], tij)
  pltpu.sync_copy(ridx_hbm.at[wid], rij)
  for j in range(NCHUNK):
    ch = pltpu.async_copy(ent_hbm.at[hij.at[j]], hbuf, sem)
    ct = pltpu.async_copy(ent_hbm.at[tij.at[j]], tbuf, sem)
    cr = pltpu.async_copy(rel_hbm.at[rij.at[j]], rbuf, sem)
    ch.wait()
    ct.wait()
    cr.wait()

    def row(i, carry):
      for d in range(DIM // LANES):
        sl = pl.ds(d * LANES, LANES)
        hbuf[i, sl] = hbuf[i, sl] + rbuf[i, sl] - tbuf[i, sl]
      return carry

    lax.fori_loop(0, CHUNK, row, 0)
    pltpu.sync_copy(hbuf, out_hbm.at[pl.ds(wid * B_PER_W + j * CHUNK, CHUNK)])


@functools.partial(
    pl.kernel,
    out_type=jax.ShapeDtypeStruct((BATCH, DIM), jnp.float32),
    mesh=plsc.VectorSubcoreMesh(core_axis_name="c", subcore_axis_name="s"),
    compiler_params=pltpu.CompilerParams(use_tc_tiling_on_sc=False),
    scratch_types=[
        pltpu.VMEM((NCHUNK, CHUNK), jnp.int32),
        pltpu.VMEM((NCHUNK, CHUNK), jnp.int32),
        pltpu.VMEM((NCHUNK, CHUNK), jnp.int32),
        pltpu.VMEM((CHUNK, DIM), jnp.float32),
        pltpu.VMEM((CHUNK, DIM), jnp.float32),
        pltpu.VMEM((CHUNK, DIM), jnp.float32),
        pltpu.SemaphoreType.DMA,
    ],
)
def _transe_sc(ent_hbm, rel_hbm, hidx_hbm, tidx_hbm, ridx_hbm, out_hbm,
               hij, tij, rij, hbuf, tbuf, rbuf, sem):
  _body(ent_hbm, rel_hbm, hidx_hbm, tidx_hbm, ridx_hbm, out_hbm,
        hij, tij, rij, hbuf, tbuf, rbuf, sem)


def kernel(ent_table, rel_table, h_list, t_list, r_list):
  h = h_list.astype(jnp.int32).reshape(NW, NCHUNK, CHUNK)
  t = t_list.astype(jnp.int32).reshape(NW, NCHUNK, CHUNK)
  r = r_list.astype(jnp.int32).reshape(NW, NCHUNK, CHUNK)
  return _transe_sc(ent_table, rel_table, h, t, r)


# speedup vs baseline: 1.5773x; 1.5773x over previous
import functools

import jax
import jax.numpy as jnp
from jax import lax
from jax.experimental import pallas as pl
from jax.experimental.pallas import tpu as pltpu
from jax.experimental.pallas import tpu_sc as plsc

DIM = 64
BATCH = 16384
LANES = 16
SUB = 8
NC = 2
NS = 16
NW = NC * NS
B_PER_W = BATCH // NW
CHUNK = 64
NCHUNK = B_PER_W // CHUNK
VPC = CHUNK // LANES


def _body(ent_hbm, rel_hbm, hidx_hbm, tidx_hbm, ridx_hbm, out_hbm,
          hiv, tiv, riv, hbuf, tbuf, rbuf, outb, hsem, tsem, rsem):
  wid = lax.axis_index("s") * NC + lax.axis_index("c")
  out3 = out_hbm.reshape(BATCH // SUB, SUB, DIM)
  pltpu.sync_copy(hidx_hbm.at[wid], hiv)
  pltpu.sync_copy(tidx_hbm.at[wid], tiv)
  pltpu.sync_copy(ridx_hbm.at[wid], riv)

  def chunk(j, carry):
    copies = []
    for k in range(VPC):
      v = j * VPC + k
      a = lax.div(v, SUB)
      b = lax.rem(v, SUB) * LANES
      hv = hiv[a, pl.ds(b, LANES)]
      tv = tiv[a, pl.ds(b, LANES)]
      rv = riv[a, pl.ds(b, LANES)]
      for l in range(LANES):
        i = k * LANES + l
        copies.append(pltpu.async_copy(ent_hbm.at[hv[l]], hbuf.at[i], hsem))
        copies.append(pltpu.async_copy(ent_hbm.at[tv[l]], tbuf.at[i], tsem))
        copies.append(pltpu.async_copy(rel_hbm.at[rv[l]], rbuf.at[i], rsem))
    for c in copies:
      c.wait()
    for i in range(CHUNK):
      ob, orow = divmod(i, SUB)
      for d in range(DIM // LANES):
        sl = pl.ds(d * LANES, LANES)
        outb[ob, orow, sl] = hbuf[i, sl] + rbuf[i, sl] - tbuf[i, sl]
    pltpu.sync_copy(outb,
                    out3.at[pl.ds(wid * (B_PER_W // SUB) + j * (CHUNK // SUB),
                                  CHUNK // SUB)])
    return carry

  lax.fori_loop(0, NCHUNK, chunk, 0)


@functools.partial(
    pl.kernel,
    out_type=jax.ShapeDtypeStruct((BATCH, DIM), jnp.float32),
    mesh=plsc.VectorSubcoreMesh(core_axis_name="c", subcore_axis_name="s"),
    compiler_params=pltpu.CompilerParams(use_tc_tiling_on_sc=True),
    scratch_types=[
        pltpu.VMEM((VPC, 128), jnp.int32),
        pltpu.VMEM((VPC, 128), jnp.int32),
        pltpu.VMEM((VPC, 128), jnp.int32),
        pltpu.VMEM((CHUNK, DIM), jnp.float32),
        pltpu.VMEM((CHUNK, DIM), jnp.float32),
        pltpu.VMEM((CHUNK, DIM), jnp.float32),
        pltpu.VMEM((CHUNK // SUB, SUB, DIM), jnp.float32),
        pltpu.SemaphoreType.DMA,
        pltpu.SemaphoreType.DMA,
        pltpu.SemaphoreType.DMA,
    ],
)
def _transe_sc(ent_hbm, rel_hbm, hidx_hbm, tidx_hbm, ridx_hbm, out_hbm,
               hiv, tiv, riv, hbuf, tbuf, rbuf, outb, hsem, tsem, rsem):
  _body(ent_hbm, rel_hbm, hidx_hbm, tidx_hbm, ridx_hbm, out_hbm,
        hiv, tiv, riv, hbuf, tbuf, rbuf, outb, hsem, tsem, rsem)


def kernel(ent_table, rel_table, h_list, t_list, r_list):
  h = h_list.astype(jnp.int32).reshape(NW, VPC, 128)
  t = t_list.astype(jnp.int32).reshape(NW, VPC, 128)
  r = r_list.astype(jnp.int32).reshape(NW, VPC, 128)
  return _transe_sc(ent_table, rel_table, h, t, r)


# speedup vs baseline: 1.5774x; 1.0001x over previous
import functools

import jax
import jax.numpy as jnp
from jax import lax
from jax.experimental import pallas as pl
from jax.experimental.pallas import tpu as pltpu
from jax.experimental.pallas import tpu_sc as plsc

DIM = 64
PAD = 128
BATCH = 16384
LANES = 16
SUB = 8
NC = 2
NS = 16
NW = NC * NS
B_PER_W = BATCH // NW
CHUNK = 128
NCHUNK = B_PER_W // CHUNK
VPC = CHUNK // LANES


def _body(ent_hbm, relp_hbm, hidx_hbm, tidx_hbm, ridx_hbm, out_hbm,
          hiv, tiv, riv, hbuf, tbuf, rbuf, outb, hsem, tsem, rsem):
  wid = lax.axis_index("s") * NC + lax.axis_index("c")
  out3 = out_hbm.reshape(BATCH // SUB, SUB, DIM)
  pltpu.sync_copy(hidx_hbm.at[wid], hiv)
  pltpu.sync_copy(tidx_hbm.at[wid], tiv)
  pltpu.sync_copy(ridx_hbm.at[wid], riv)

  def chunk(j, carry):
    copies = [pltpu.async_copy(relp_hbm.at[riv.at[j]], rbuf, rsem)]
    for k in range(VPC):
      sl = pl.ds(k * LANES, LANES)
      hv = hiv[j, sl]
      tv = tiv[j, sl]
      for l in range(LANES):
        i = k * LANES + l
        copies.append(pltpu.async_copy(ent_hbm.at[hv[l]], hbuf.at[i], hsem))
        copies.append(pltpu.async_copy(ent_hbm.at[tv[l]], tbuf.at[i], tsem))
    for c in copies:
      c.wait()

    for i in range(CHUNK):
      ob, orow = divmod(i, SUB)
      for d in range(DIM // LANES):
        sl = pl.ds(d * LANES, LANES)
        outb[ob, orow, sl] = hbuf[i, sl] + rbuf[i, sl] - tbuf[i, sl]
    pltpu.sync_copy(outb,
                    out3.at[pl.ds(wid * (B_PER_W // SUB) + j * (CHUNK // SUB),
                                  CHUNK // SUB)])
    return carry

  lax.fori_loop(0, NCHUNK, chunk, 0)


@functools.partial(
    pl.kernel,
    out_type=jax.ShapeDtypeStruct((BATCH, DIM), jnp.float32),
    mesh=plsc.VectorSubcoreMesh(core_axis_name="c", subcore_axis_name="s"),
    compiler_params=pltpu.CompilerParams(use_tc_tiling_on_sc=True),
    scratch_types=[
        pltpu.VMEM((NCHUNK, CHUNK), jnp.int32),
        pltpu.VMEM((NCHUNK, CHUNK), jnp.int32),
        pltpu.VMEM((NCHUNK, CHUNK), jnp.int32),
        pltpu.VMEM((CHUNK, DIM), jnp.float32),
        pltpu.VMEM((CHUNK, DIM), jnp.float32),
        pltpu.VMEM((CHUNK, PAD), jnp.float32),
        pltpu.VMEM((CHUNK // SUB, SUB, DIM), jnp.float32),
        pltpu.SemaphoreType.DMA,
        pltpu.SemaphoreType.DMA,
        pltpu.SemaphoreType.DMA,
    ],
)
def _transe_sc(ent_hbm, relp_hbm, hidx_hbm, tidx_hbm, ridx_hbm, out_hbm,
               hiv, tiv, riv, hbuf, tbuf, rbuf, outb, hsem, tsem, rsem):
  _body(ent_hbm, relp_hbm, hidx_hbm, tidx_hbm, ridx_hbm, out_hbm,
        hiv, tiv, riv, hbuf, tbuf, rbuf, outb, hsem, tsem, rsem)


def kernel(ent_table, rel_table, h_list, t_list, r_list):
  relp = jnp.pad(rel_table, ((0, 0), (0, PAD - DIM)))
  h = h_list.astype(jnp.int32).reshape(NW, NCHUNK, CHUNK)
  t = t_list.astype(jnp.int32).reshape(NW, NCHUNK, CHUNK)
  r = r_list.astype(jnp.int32).reshape(NW, NCHUNK, CHUNK)
  return _transe_sc(ent_table, relp, h, t, r)
